# baseline (speedup 1.0000x reference)
import jax
import jax.numpy as jnp
from jax.experimental import pallas as pl
from jax.experimental.pallas import tpu as pltpu
from jax.experimental.pallas import tpu_sc as plsc

_N = 20000
_TOPK = 5000
_KEEP = 750
_CONF = 0.8
_T = 0.3
_B = 512
_NB = 10
_NP = _B * _NB
_R, _C = 256, 128
_NT = _R * _C
_NSLAB = 32
_GW = 128
_VW = 128
_OUTP = 768


def _local_stages(s, idx, i, k_lo, k_hi):
    sub = jax.lax.broadcasted_iota(jnp.int32, (8, _C), 0)
    lane = jax.lax.broadcasted_iota(jnp.int32, (8, _C), 1)
    e = sub * _C + lane

    lk = 1
    while (1 << lk) <= k_hi:
        k = 1 << lk
        if k >= k_lo:
            for lj in range(min(lk - 1, 9), -1, -1):
                j = 1 << lj
                if j < _C:
                    axis, sh = 1, j
                    lower = (lane & j) == 0
                else:
                    axis, sh = 0, j // _C
                    lower = (sub & sh) == 0
                ps = jnp.where(lower, jnp.roll(s, -sh, axis=axis),
                               jnp.roll(s, sh, axis=axis))
                pi = jnp.where(lower, jnp.roll(idx, -sh, axis=axis),
                               jnp.roll(idx, sh, axis=axis))
                better = (s > ps) | ((s == ps) & (idx < pi))
                if k <= 512:
                    asc = (e & k) == 0
                else:
                    asc = ((i * 1024) & k) == 0
                take_own = better == (lower == asc)
                s = jnp.where(take_own, s, ps)
                idx = jnp.where(take_own, idx, pi)
        lk += 1
    return s, idx


def _sort_body(s_in, s_ref, i_ref):
    def phase_a(i, carry):
        sub = jax.lax.broadcasted_iota(jnp.int32, (8, _C), 0)
        lane = jax.lax.broadcasted_iota(jnp.int32, (8, _C), 1)
        e = sub * _C + lane
        p = i * 1024 + e
        s = s_in[pl.ds(i * 8, 8), :]
        s = jnp.where(s >= _CONF, s, -1.0)
        s = jnp.where(p < _N, s, -2.0)
        s, idx = _local_stages(s, p, i, 2, 1024)
        s_ref[pl.ds(i * 8, 8), :] = s
        i_ref[pl.ds(i * 8, 8), :] = idx
        return carry

    jax.lax.fori_loop(0, _NSLAB, phase_a, 0)

    for lk in range(11, 16):
        k = 1 << lk
        for lj in range(lk - 1, 9, -1):
            J = 1 << (lj - 10)

            def cross(m, carry, J=J, k=k):
                a = (m // J) * (2 * J) + (m % J)
                b = a + J
                asc = ((a * 1024) & k) == 0
                sa = s_ref[pl.ds(a * 8, 8), :]
                sb = s_ref[pl.ds(b * 8, 8), :]
                ia = i_ref[pl.ds(a * 8, 8), :]
                ib = i_ref[pl.ds(b * 8, 8), :]
                win_a = (sa > sb) | ((sa == sb) & (ia < ib))
                sel_a = win_a == asc
                s_ref[pl.ds(a * 8, 8), :] = jnp.where(sel_a, sa, sb)
                i_ref[pl.ds(a * 8, 8), :] = jnp.where(sel_a, ia, ib)
                sel_b = jnp.logical_not(sel_a)
                s_ref[pl.ds(b * 8, 8), :] = jnp.where(sel_b, sa, sb)
                i_ref[pl.ds(b * 8, 8), :] = jnp.where(sel_b, ia, ib)
                return carry

            jax.lax.fori_loop(0, _NSLAB // 2, cross, 0)

        def local_b(i, carry, k=k):
            s = s_ref[pl.ds(i * 8, 8), :]
            idx = i_ref[pl.ds(i * 8, 8), :]
            s, idx = _local_stages(s, idx, i, k, k)
            s_ref[pl.ds(i * 8, 8), :] = s
            i_ref[pl.ds(i * 8, 8), :] = idx
            return carry

        jax.lax.fori_loop(0, _NSLAB, local_b, 0)


def _sort_call(scores_pad):
    return pl.pallas_call(
        _sort_body,
        out_shape=(jax.ShapeDtypeStruct((_R, _C), jnp.float32),
                   jax.ShapeDtypeStruct((_R, _C), jnp.int32)),
    )(scores_pad)


def _nms_body(xr, yr, Xr, Yr, sr, bc, dest_ref, ksc_ref, keep_ref):
    i = pl.program_id(0)
    f32 = jnp.float32

    @pl.when(i == 0)
    def _init():
        keep_ref[...] = (sr[...] > 0.0).astype(f32)

    base = i * _B
    px1 = bc[pl.ds(base, _B), 0:1]
    py1 = bc[pl.ds(base, _B), 1:2]
    px2 = bc[pl.ds(base, _B), 2:3]
    py2 = bc[pl.ds(base, _B), 3:4]
    pa = jnp.maximum(px2 - px1, 0.0) * jnp.maximum(py2 - py1, 0.0)

    def strip_sup(j):
        tx1 = xr[pl.ds(j, 1), :]
        ty1 = yr[pl.ds(j, 1), :]
        tx2 = Xr[pl.ds(j, 1), :]
        ty2 = Yr[pl.ds(j, 1), :]
        ta = jnp.maximum(tx2 - tx1, 0.0) * jnp.maximum(ty2 - ty1, 0.0)
        ix1 = jnp.maximum(px1, tx1)
        iy1 = jnp.maximum(py1, ty1)
        ix2 = jnp.minimum(px2, tx2)
        iy2 = jnp.minimum(py2, ty2)
        iw = jnp.maximum(ix2 - ix1, 0.0)
        ih = jnp.maximum(iy2 - iy1, 0.0)
        inter = iw * ih
        union = (pa + ta) - inter
        iou = inter / jnp.maximum(union, 1e-9)
        return (iou > _T).astype(f32)

    sup_ii = strip_sup(i)
    ci = jax.lax.broadcasted_iota(jnp.int32, (_B, _B), 1)
    ri = jax.lax.broadcasted_iota(jnp.int32, (_B, _B), 0)
    sup_ii = jnp.where(ci > ri, sup_ii, 0.0)

    k0 = keep_ref[pl.ds(i, 1), :]

    def cond(st):
        return st[1]

    def body(st):
        k, _ = st
        kc = jnp.transpose(k, (1, 0))
        supp = jnp.max(sup_ii * kc, axis=0, keepdims=True)
        kn = jnp.where(supp > 0.0, 0.0, k0)
        return kn, jnp.any(kn != k)

    kfin, _ = jax.lax.while_loop(cond, body, (k0, jnp.bool_(True)))
    keep_ref[pl.ds(i, 1), :] = kfin
    kfc = jnp.transpose(kfin, (1, 0))

    def cross(j, carry):
        sup = strip_sup(j)
        supp = jnp.max(sup * kfc, axis=0, keepdims=True)
        kj = keep_ref[pl.ds(j, 1), :]
        keep_ref[pl.ds(j, 1), :] = jnp.where(supp > 0.0, 0.0, kj)
        return carry

    jax.lax.fori_loop(i + 1, _NB, cross, 0)

    @pl.when(i == _NB - 1)
    def _fin():
        keep = keep_ref[...]
        r2 = jax.lax.broadcasted_iota(jnp.int32, (_NB, _B), 0)
        c2 = jax.lax.broadcasted_iota(jnp.int32, (_NB, _B), 1)
        pidx = r2 * _B + c2
        real = jnp.where(pidx < _TOPK, 1.0, 0.0)
        nonk = (1.0 - keep) * real

        r1 = jax.lax.broadcasted_iota(jnp.int32, (_NB, 1), 0)

        def cumsum_linear(m):
            x = m
            sh = 1
            while sh < _B:
                x = x + jnp.where(c2 >= sh, jnp.roll(x, sh, axis=1), 0.0)
                sh *= 2
            tot = x[:, _B - 1:_B]
            off = tot
            sh = 1
            while sh < _NB:
                off = off + jnp.where(r1 >= sh, jnp.roll(off, sh, axis=0), 0.0)
                sh *= 2
            return x + (off - tot)

        ck = cumsum_linear(keep)
        cn = cumsum_linear(nonk)
        nk = ck[_NB - 1:_NB, _B - 1:_B]
        dest = jnp.where(keep > 0.0, ck - 1.0, (cn - 1.0) + nk)
        ok = ((keep + nonk) > 0.0) & (dest < float(_KEEP))
        dump = (752 + (pidx & 15)).astype(f32)
        dest_ref[...] = jnp.where(ok, dest, dump).astype(jnp.int32)
        ksc_ref[...] = jnp.where(keep > 0.0, sr[...], -1.0)


def _nms_call(xr, yr, Xr, Yr, sr, bc):
    full = lambda i: (0, 0)
    return pl.pallas_call(
        _nms_body,
        grid=(_NB,),
        in_specs=[
            pl.BlockSpec((_NB, _B), full),
            pl.BlockSpec((_NB, _B), full),
            pl.BlockSpec((_NB, _B), full),
            pl.BlockSpec((_NB, _B), full),
            pl.BlockSpec((_NB, _B), full),
            pl.BlockSpec((_NP, 8), full),
        ],
        out_specs=(pl.BlockSpec((_NB, _B), full),
                   pl.BlockSpec((_NB, _B), full)),
        out_shape=(jax.ShapeDtypeStruct((_NB, _B), jnp.int32),
                   jax.ShapeDtypeStruct((_NB, _B), jnp.float32)),
        scratch_shapes=[pltpu.VMEM((_NB, _B), jnp.float32)],
    )(xr, yr, Xr, Yr, sr, bc)


def _sc_mesh():
    return plsc.VectorSubcoreMesh(core_axis_name="core",
                                  subcore_axis_name="subcore")


def _sc_gather(x128, idx2d):
    @pl.kernel(out_type=jax.ShapeDtypeStruct((_NP, _VW), jnp.float32),
               mesh=_sc_mesh())
    def k(x_hbm, i_hbm, o_hbm):
        def body(i_vmem, o_vmem):
            pltpu.sync_copy(x_hbm.at[i_vmem.at[0]], o_vmem)

        pltpu.emit_pipeline(
            body,
            grid=(_NP // _GW,),
            in_specs=[pl.BlockSpec((1, _GW), index_map=lambda i: (0, i))],
            out_specs=[pl.BlockSpec((_GW, _VW), index_map=lambda i: (i, 0))],
            core_axis_name=("core", "subcore"),
            dimension_semantics=(pltpu.PARALLEL,),
        )(i_hbm, o_hbm)

    return k(x128, idx2d)


def _sc_scatter(data128, dest2d):
    @pl.kernel(out_type=jax.ShapeDtypeStruct((_OUTP, _VW), jnp.float32),
               mesh=_sc_mesh())
    def k(x_hbm, i_hbm, o_hbm):
        def body(x_vmem, i_vmem):
            pltpu.sync_copy(x_vmem, o_hbm.at[i_vmem.at[0]])

        pltpu.emit_pipeline(
            body,
            grid=(_NP // _GW,),
            in_specs=[pl.BlockSpec((_GW, _VW), index_map=lambda i: (i, 0)),
                      pl.BlockSpec((1, _GW), index_map=lambda i: (0, i))],
            out_specs=[],
            core_axis_name=("core", "subcore"),
            dimension_semantics=(pltpu.PARALLEL,),
        )(x_hbm, i_hbm)

    return k(data128, dest2d)


@jax.jit
def kernel(boxes, scores):
    f32 = jnp.float32
    spad = jnp.concatenate([scores, jnp.zeros((_NT - _N,), f32)]).reshape(_R, _C)
    ss, si = _sort_call(spad)
    ssf = ss.reshape(_NT)[:_NP]
    sif = si.reshape(_NT)[:_NP]

    boxes128 = jnp.concatenate([boxes, jnp.zeros((_N, _VW - 4), f32)], 1)
    g = _sc_gather(boxes128, sif.reshape(1, _NP))

    xr = g[:, 0].reshape(_NB, _B)
    yr = g[:, 1].reshape(_NB, _B)
    Xr = g[:, 2].reshape(_NB, _B)
    Yr = g[:, 3].reshape(_NB, _B)
    sr = ssf.reshape(_NB, _B)
    bc = jnp.concatenate([g[:, :4], ssf[:, None], jnp.zeros((_NP, 3), f32)], 1)
    dest, ksc = _nms_call(xr, yr, Xr, Yr, sr, bc)

    data128 = jnp.concatenate(
        [g[:, :4], ksc.reshape(_NP)[:, None], jnp.zeros((_NP, _VW - 5), f32)], 1)
    out128 = _sc_scatter(data128, dest.reshape(1, _NP))
    return out128[:_KEEP, :5]


# speedup vs baseline: 3.1879x; 3.1879x over previous
import jax
import jax.numpy as jnp
from jax.experimental import pallas as pl
from jax.experimental.pallas import tpu as pltpu
from jax.experimental.pallas import tpu_sc as plsc

_N = 20000
_TOPK = 5000
_KEEP = 750
_CONF = 0.8
_T = 0.3
_B = 512
_NB = 10
_NP = _B * _NB
_R, _C = 256, 128
_NT = _R * _C
_NSLAB = 32
_GW = 128
_VW = 128
_OUTP = 768


_SUB = None


def _stage_step(s, idx, i, k, j, sub, lane, e):
    if j < _C:
        axis, sh = 1, j
        lower = (lane & j) == 0
    else:
        axis, sh = 0, j // _C
        lower = (sub & sh) == 0
    ps = jnp.where(lower, jnp.roll(s, -sh, axis=axis),
                   jnp.roll(s, sh, axis=axis))
    pi = jnp.where(lower, jnp.roll(idx, -sh, axis=axis),
                   jnp.roll(idx, sh, axis=axis))
    better = (s > ps) | ((s == ps) & (idx < pi))
    if k <= 512:
        asc = (e & k) == 0
        take_own = better == (lower == asc)
    else:
        asc = ((i * 1024) & k) == 0
        take_own = better == (lower if asc else jnp.logical_not(lower))
    s = jnp.where(take_own, s, ps)
    idx = jnp.where(take_own, idx, pi)
    return s, idx


def _run_stages_batch(slabs, stages, sub, lane, e):
    for (k, j) in stages:
        slabs = [(i,) + _stage_step(s, idx, i, k, j, sub, lane, e)
                 for (i, s, idx) in slabs]
    return slabs


def _sort_body(s_in, s_ref, i_ref):
    sub = jax.lax.broadcasted_iota(jnp.int32, (8, _C), 0)
    lane = jax.lax.broadcasted_iota(jnp.int32, (8, _C), 1)
    e = sub * _C + lane
    n_real = (_N + 1023) // 1024
    G = 32

    stages_a = [(1 << lk, 1 << lj)
                for lk in range(1, 11) for lj in range(lk - 1, -1, -1)]

    for b0 in range(0, n_real, G):
        batch = []
        for i in range(b0, min(b0 + G, n_real)):
            p = i * 1024 + e
            s = s_in[i * 8:(i + 1) * 8, :]
            s = jnp.where(s >= _CONF, s, -1.0)
            s = jnp.where(p < _N, s, -2.0)
            batch.append((i, s, p))
        batch = _run_stages_batch(batch, stages_a, sub, lane, e)
        for (i, s, idx) in batch:
            s_ref[i * 8:(i + 1) * 8, :] = s
            i_ref[i * 8:(i + 1) * 8, :] = idx

    for i in range(n_real, _NSLAB):
        s_ref[i * 8:(i + 1) * 8, :] = jnp.full((8, _C), -2.0, jnp.float32)
        i_ref[i * 8:(i + 1) * 8, :] = i * 1024 + (e if i % 2 == 0 else 1023 - e)

    for lk in range(11, 16):
        k = 1 << lk
        for lj in range(lk - 1, 9, -1):
            J = 1 << (lj - 10)
            pairs = [( (m // J) * (2 * J) + (m % J),
                       (m // J) * (2 * J) + (m % J) + J)
                     for m in range(_NSLAB // 2)]
            for g0 in range(0, len(pairs), G):
                loaded = []
                for (a, bb) in pairs[g0:g0 + G]:
                    loaded.append((a, bb,
                                   s_ref[a * 8:(a + 1) * 8, :],
                                   s_ref[bb * 8:(bb + 1) * 8, :],
                                   i_ref[a * 8:(a + 1) * 8, :],
                                   i_ref[bb * 8:(bb + 1) * 8, :]))
                for (a, bb, sa, sb, ia, ib) in loaded:
                    asc = ((a * 1024) & k) == 0
                    win_a = (sa > sb) | ((sa == sb) & (ia < ib))
                    sel_a = win_a if asc else jnp.logical_not(win_a)
                    sel_b = jnp.logical_not(sel_a)
                    s_ref[a * 8:(a + 1) * 8, :] = jnp.where(sel_a, sa, sb)
                    i_ref[a * 8:(a + 1) * 8, :] = jnp.where(sel_a, ia, ib)
                    s_ref[bb * 8:(bb + 1) * 8, :] = jnp.where(sel_b, sa, sb)
                    i_ref[bb * 8:(bb + 1) * 8, :] = jnp.where(sel_b, ia, ib)

        stages_k = [(k, 1 << lj) for lj in range(9, -1, -1)]
        for b0 in range(0, _NSLAB, G):
            batch = [(i,
                      s_ref[i * 8:(i + 1) * 8, :],
                      i_ref[i * 8:(i + 1) * 8, :])
                     for i in range(b0, b0 + G)]
            batch = _run_stages_batch(batch, stages_k, sub, lane, e)
            for (i, s, idx) in batch:
                s_ref[i * 8:(i + 1) * 8, :] = s
                i_ref[i * 8:(i + 1) * 8, :] = idx


def _sort_call(scores_pad):
    return pl.pallas_call(
        _sort_body,
        out_shape=(jax.ShapeDtypeStruct((_R, _C), jnp.float32),
                   jax.ShapeDtypeStruct((_R, _C), jnp.int32)),
    )(scores_pad)


def _nms_body(xr, yr, Xr, Yr, sr, bc, dest_ref, ksc_ref, keep_ref):
    f32 = jnp.float32
    keep_ref[...] = jnp.zeros((_NB, _B), f32)

    ci = jax.lax.broadcasted_iota(jnp.int32, (_B, _B), 1)
    ri = jax.lax.broadcasted_iota(jnp.int32, (_B, _B), 0)

    def pivot_step(st):
        i, cnt = st
        base = i * _B
        px1 = bc[pl.ds(base, _B), 0:1]
        py1 = bc[pl.ds(base, _B), 1:2]
        px2 = bc[pl.ds(base, _B), 2:3]
        py2 = bc[pl.ds(base, _B), 3:4]
        psc = bc[pl.ds(base, _B), 4:5]
        pa = jnp.maximum(px2 - px1, 0.0) * jnp.maximum(py2 - py1, 0.0)

        def strip(j):
            tx1 = xr[pl.ds(j, 1), :]
            ty1 = yr[pl.ds(j, 1), :]
            tx2 = Xr[pl.ds(j, 1), :]
            ty2 = Yr[pl.ds(j, 1), :]
            ta = jnp.maximum(tx2 - tx1, 0.0) * jnp.maximum(ty2 - ty1, 0.0)
            ix1 = jnp.maximum(px1, tx1)
            iy1 = jnp.maximum(py1, ty1)
            ix2 = jnp.minimum(px2, tx2)
            iy2 = jnp.minimum(py2, ty2)
            iw = jnp.maximum(ix2 - ix1, 0.0)
            ih = jnp.maximum(iy2 - iy1, 0.0)
            inter = iw * ih
            union = (pa + ta) - inter
            iou = inter / jnp.maximum(union, 1e-9)
            return (iou > _T).astype(f32)

        def incoming(j, kcol):
            m = strip(j)
            kj = keep_ref[pl.ds(j, 1), :]
            supp = jnp.max(m * kj, axis=1, keepdims=True)
            return jnp.where(supp > 0.0, 0.0, kcol)

        k0 = jax.lax.fori_loop(0, i, incoming, (psc > 0.0).astype(f32))

        sup_low = jnp.where(ci < ri, strip(i), 0.0)

        def cond(s):
            return s[1]

        def body(s):
            k, _ = s
            krow = jnp.transpose(k, (1, 0))
            supp = jnp.max(sup_low * krow, axis=1, keepdims=True)
            kn = jnp.where(supp > 0.0, 0.0, k0)
            return kn, jnp.any(kn != k)

        kfin, _ = jax.lax.while_loop(cond, body, (k0, jnp.bool_(True)))
        keep_ref[pl.ds(i, 1), :] = jnp.transpose(kfin, (1, 0))
        return i + 1, cnt + jnp.sum(kfin)

    def pivot_cond(st):
        i, cnt = st
        return (i < _NB) & (cnt < float(_KEEP))

    jax.lax.while_loop(pivot_cond, pivot_step, (0, jnp.float32(0.0)))

    keep = keep_ref[...]
    r2 = jax.lax.broadcasted_iota(jnp.int32, (_NB, _B), 0)
    c2 = jax.lax.broadcasted_iota(jnp.int32, (_NB, _B), 1)
    pidx = r2 * _B + c2
    real = jnp.where(pidx < _TOPK, 1.0, 0.0)
    nonk = (1.0 - keep) * real

    r1 = jax.lax.broadcasted_iota(jnp.int32, (_NB, 1), 0)

    def cumsum_linear(m):
        x = m
        sh = 1
        while sh < _B:
            x = x + jnp.where(c2 >= sh, jnp.roll(x, sh, axis=1), 0.0)
            sh *= 2
        tot = x[:, _B - 1:_B]
        off = tot
        sh = 1
        while sh < _NB:
            off = off + jnp.where(r1 >= sh, jnp.roll(off, sh, axis=0), 0.0)
            sh *= 2
        return x + (off - tot)

    ck = cumsum_linear(keep)
    cn = cumsum_linear(nonk)
    nk = ck[_NB - 1:_NB, _B - 1:_B]
    dest = jnp.where(keep > 0.0, ck - 1.0, (cn - 1.0) + nk)
    ok = ((keep + nonk) > 0.0) & (dest < float(_KEEP))
    dump = (752 + (pidx & 15)).astype(f32)
    dest_ref[...] = jnp.where(ok, dest, dump).astype(jnp.int32)
    ksc_ref[...] = jnp.where(keep > 0.0, sr[...], -1.0)


def _nms_call(xr, yr, Xr, Yr, sr, bc):
    return pl.pallas_call(
        _nms_body,
        out_shape=(jax.ShapeDtypeStruct((_NB, _B), jnp.int32),
                   jax.ShapeDtypeStruct((_NB, _B), jnp.float32)),
        scratch_shapes=[pltpu.VMEM((_NB, _B), jnp.float32)],
    )(xr, yr, Xr, Yr, sr, bc)


def _sc_mesh():
    return plsc.VectorSubcoreMesh(core_axis_name="core",
                                  subcore_axis_name="subcore")


def _sc_gather(x128, idx2d):
    @pl.kernel(out_type=jax.ShapeDtypeStruct((_NP, _VW), jnp.float32),
               mesh=_sc_mesh())
    def k(x_hbm, i_hbm, o_hbm):
        def body(i_vmem, o_vmem):
            pltpu.sync_copy(x_hbm.at[i_vmem.at[0]], o_vmem)

        pltpu.emit_pipeline(
            body,
            grid=(_NP // _GW,),
            in_specs=[pl.BlockSpec((1, _GW), index_map=lambda i: (0, i))],
            out_specs=[pl.BlockSpec((_GW, _VW), index_map=lambda i: (i, 0))],
            core_axis_name=("core", "subcore"),
            dimension_semantics=(pltpu.PARALLEL,),
        )(i_hbm, o_hbm)

    return k(x128, idx2d)


def _sc_scatter(data128, dest2d):
    @pl.kernel(out_type=jax.ShapeDtypeStruct((_OUTP, _VW), jnp.float32),
               mesh=_sc_mesh())
    def k(x_hbm, i_hbm, o_hbm):
        def body(x_vmem, i_vmem):
            pltpu.sync_copy(x_vmem, o_hbm.at[i_vmem.at[0]])

        pltpu.emit_pipeline(
            body,
            grid=(_NP // _GW,),
            in_specs=[pl.BlockSpec((_GW, _VW), index_map=lambda i: (i, 0)),
                      pl.BlockSpec((1, _GW), index_map=lambda i: (0, i))],
            out_specs=[],
            core_axis_name=("core", "subcore"),
            dimension_semantics=(pltpu.PARALLEL,),
        )(x_hbm, i_hbm)

    return k(data128, dest2d)


@jax.jit
def kernel(boxes, scores):
    f32 = jnp.float32
    spad = jnp.concatenate([scores, jnp.zeros((_NT - _N,), f32)]).reshape(_R, _C)
    ss, si = _sort_call(spad)
    ssf = ss.reshape(_NT)[:_NP]
    sif = si.reshape(_NT)[:_NP]

    boxes128 = jnp.concatenate([boxes, jnp.zeros((_N, _VW - 4), f32)], 1)
    g = _sc_gather(boxes128, sif.reshape(1, _NP))

    xr = g[:, 0].reshape(_NB, _B)
    yr = g[:, 1].reshape(_NB, _B)
    Xr = g[:, 2].reshape(_NB, _B)
    Yr = g[:, 3].reshape(_NB, _B)
    sr = ssf.reshape(_NB, _B)
    bc = jnp.concatenate([g[:, :4], ssf[:, None], jnp.zeros((_NP, 3), f32)], 1)
    dest, ksc = _nms_call(xr, yr, Xr, Yr, sr, bc)

    data128 = jnp.concatenate(
        [g[:, :4], ksc.reshape(_NP)[:, None], jnp.zeros((_NP, _VW - 5), f32)], 1)
    out128 = _sc_scatter(data128, dest.reshape(1, _NP))
    return out128[:_KEEP, :5]
